# baseline (speedup 1.0000x reference)
import functools

import jax
import jax.numpy as jnp
from jax import lax
from jax.experimental import pallas as pl
from jax.experimental.pallas import tpu as pltpu
from jax.experimental.pallas import tpu_sc as plsc

N = 5000
C = 256
PE_IN = 64
PE_D = 32
NHOM = 10000
E = 80000
EHOM = 160000

NC, NS, L = 2, 16, 16
CH = 2048
NCHUNK = 40
SUB = 64
ACC_N = 5120
ACC_H = 10240
RT_N = ACC_N // NS
RT_H = ACC_H // NS
PAD_R = 16
PE_W = 128

_i32 = jnp.int32


def _splat(vec, l):
    idx = jnp.full((L,), l, _i32)
    return lax.gather(
        vec, idx[:, None],
        lax.GatherDimensionNumbers(
            offset_dims=(), collapsed_slice_dims=(0,), start_index_map=(0,)),
        (1,), mode=lax.GatherScatterMode.PROMISE_IN_BOUNDS)


def _make_segsum_body(width, rows_per_tile, with_counts):
    n_base = rows_per_tile + PAD_R
    n_acc = n_base + (2 if with_counts else 0)
    n_slice = width // L

    def body(x_hbm, src_hbm, dst_hbm, z_acc, *rest):
        if with_counts:
            out, cnt_out, sbuf, dbuf, s_sel, d_sel, rows, acc = rest
        else:
            out, sbuf, dbuf, s_sel, d_sel, rows, acc = rest
        c = lax.axis_index("c")
        s = lax.axis_index("s")
        lo = s * rows_per_tile
        pltpu.sync_copy(z_acc, acc)
        iota = lax.iota(_i32, L)
        lo_v = jnp.zeros((L,), _i32) + lo
        ones_f = jnp.full((L,), 1.0, jnp.float32)
        lane0 = iota < 1

        def chunk_body(j, carry):
            pltpu.sync_copy(src_hbm.at[c, j], sbuf)
            pltpu.sync_copy(dst_hbm.at[c, j], dbuf)

            def fill(i, carry2):
                s_sel[pl.ds(i * L, L)] = jnp.zeros((L,), _i32)
                d_sel[pl.ds(i * L, L)] = jnp.zeros((L,), _i32) + rows_per_tile
                return carry2
            lax.fori_loop(0, (CH + SUB) // L, fill, 0)

            def compact(g, cnt_v):
                dvec = dbuf[pl.ds(g * L, L)]
                svec = sbuf[pl.ds(g * L, L)]
                dloc = dvec - lo_v
                m = (dloc >= 0) & (dloc < rows_per_tile)
                mi = jnp.where(m, 1, 0).astype(_i32)
                inc = plsc.cumsum(mi)
                pos = cnt_v + inc - mi
                plsc.store_scatter(s_sel, [pos], svec, mask=m)
                plsc.store_scatter(d_sel, [pos], dloc, mask=m)
                return cnt_v + _splat(inc, L - 1)
            cnt_v = lax.fori_loop(0, CH // L, compact,
                                  jnp.zeros((L,), _i32))

            def sub(gsub, carry2):
                base = gsub * SUB

                @pl.when(jnp.any(cnt_v > base))
                def _():
                    pltpu.sync_copy(x_hbm.at[s_sel.at[pl.ds(base, SUB)]],
                                    rows)

                    def acc_group(k, carry3):
                        dvec = d_sel[pl.ds(base + k * L, L)]
                        for l in range(L):
                            dsp = _splat(dvec, l)
                            for mc in range(n_slice):
                                v = rows[k * L + l, pl.ds(mc * L, L)]
                                plsc.addupdate_scatter(
                                    acc, [dsp, iota + (mc * L)], v)
                            if with_counts:
                                plsc.addupdate_scatter(
                                    acc,
                                    [n_base + (dsp >> 8), dsp & (width - 1)],
                                    ones_f, mask=lane0)
                        return carry3
                    lax.fori_loop(0, SUB // L, acc_group, 0)
                return carry2
            lax.fori_loop(0, CH // SUB, sub, 0)
            return carry

        lax.fori_loop(0, NCHUNK, chunk_body, 0)
        pltpu.sync_copy(acc.at[pl.ds(0, rows_per_tile)],
                        out.at[c, pl.ds(lo, rows_per_tile)])
        if with_counts:
            pltpu.sync_copy(acc.at[pl.ds(n_base, 2)],
                            cnt_out.at[c, pl.ds(s * 2, 2)])

    scratch = [
        pltpu.VMEM((CH,), _i32),
        pltpu.VMEM((CH,), _i32),
        pltpu.VMEM((CH + SUB,), _i32),
        pltpu.VMEM((CH + SUB,), _i32),
        pltpu.VMEM((SUB, width), jnp.float32),
        pltpu.VMEM((n_acc, width), jnp.float32),
    ]
    return body, tuple(scratch)


@functools.lru_cache(maxsize=None)
def _sc_kernels():
    mesh = plsc.VectorSubcoreMesh(
        core_axis_name="c", subcore_axis_name="s",
        num_cores=NC, num_subcores=NS)
    params = pltpu.CompilerParams(needs_layout_passes=False)

    body_h, scratch_h = _make_segsum_body(PE_W, RT_H, False)
    hom = pl.kernel(
        body_h,
        out_type=jax.ShapeDtypeStruct((NC, ACC_H, PE_W), jnp.float32),
        mesh=mesh, compiler_params=params, scratch_types=scratch_h)

    body_m1, scratch_m1 = _make_segsum_body(C, RT_N, True)
    mean1 = pl.kernel(
        body_m1,
        out_type=(jax.ShapeDtypeStruct((NC, ACC_N, C), jnp.float32),
                  jax.ShapeDtypeStruct((NC, NS * 2, C), jnp.float32)),
        mesh=mesh, compiler_params=params, scratch_types=scratch_m1)

    body_m2, scratch_m2 = _make_segsum_body(C, RT_N, False)
    mean2 = pl.kernel(
        body_m2,
        out_type=jax.ShapeDtypeStruct((NC, ACC_N, C), jnp.float32),
        mesh=mesh, compiler_params=params, scratch_types=scratch_m2)
    return hom, mean1, mean2


def _sc_hom(*args):
    return _sc_kernels()[0](*args)


def _sc_mean1(*args):
    return _sc_kernels()[1](*args)


def _sc_mean2(*args):
    return _sc_kernels()[2](*args)


def _ln_relu(x, g, b):
    mu = jnp.mean(x, axis=-1, keepdims=True)
    xc = x - mu
    var = jnp.mean(xc * xc, axis=-1, keepdims=True)
    return jax.nn.relu(xc * lax.rsqrt(var + 1e-5) * g + b)


def _inv_cnt(cnt_slice):
    return 1.0 / jnp.maximum(cnt_slice, 1.0)


def _tc_dense1(pe_ref, homp_ref, xu_ref, xi_ref,
               s1_ref, gw11, gb11, gw21, gb21,
               s2_ref, gw12, gb12, gw22, gb22,
               pewx, pewp, peb,
               xall_out, pe2_out):
    aggh = homp_ref[0] + homp_ref[1]
    pe0 = pe_ref[...]
    h1 = pe0 * s1_ref[0, 0] + aggh
    pe1 = jax.nn.relu(h1 @ gw11[...] + gb11[...]) @ gw21[...] + gb21[...]
    h2 = pe0 * s2_ref[0, 0] + aggh
    pe2 = jax.nn.relu(h2 @ gw12[...] + gb12[...]) @ gw22[...] + gb22[...]
    pe2_out[...] = pe2
    xall_out[:N] = xu_ref[...] @ pewx[...] + pe1[:N] @ pewp[...] + peb[...]
    xall_out[N:] = xi_ref[...] @ pewx[...] + pe1[N:] @ pewp[...] + peb[...]


def _tc_sage(sums_ref, cnt_ref, xall_ref,
             wl_u2i, bl_u2i, wr_u2i, wl_i2u, bl_i2u, wr_i2u,
             lng_u, lnb_u, lng_i, lnb_i,
             out_ref):
    xu = xall_ref[:N]
    xi = xall_ref[N:]
    agg_i = sums_ref[0] * _inv_cnt(cnt_ref[0])
    nxi = agg_i @ wl_u2i[...] + bl_u2i[...] + xi @ wr_u2i[...]
    out_ref[N:] = _ln_relu(nxi, lng_i[...], lnb_i[...])
    agg_u = sums_ref[1] * _inv_cnt(cnt_ref[1])
    nxu = agg_u @ wl_i2u[...] + bl_i2u[...] + xu @ wr_i2u[...]
    out_ref[:N] = _ln_relu(nxu, lng_u[...], lnb_u[...])


def _tc_pew(y_ref, pe2_ref, pewx2, pewp2, peb2, xall2_out):
    xall2_out[...] = (y_ref[...] @ pewx2[...] + pe2_ref[...] @ pewp2[...]
                      + peb2[...])


def _pad_idx(a, total, fill):
    return jnp.concatenate(
        [a, jnp.full((total - a.shape[0],), fill, _i32)])


def _pad_idx_spread(a, total, lo, hi):
    n = total - a.shape[0]
    return jnp.concatenate(
        [a, lo + (jnp.arange(n, dtype=_i32) % (hi - lo))])


def _row(v):
    return jnp.reshape(v, (1, -1))


def kernel(x_user, x_item, PE, edge_index_u2i, edge_index_i2u,
           edge_index_hom, params):
    l1, l2 = params['layers']

    per_sc = NCHUNK * CH
    src0 = _pad_idx(edge_index_u2i[0], per_sc, 0)
    dst0 = _pad_idx_spread(edge_index_u2i[1], per_sc, N, ACC_N)
    src1 = _pad_idx(edge_index_i2u[0], per_sc, 0) + N
    dst1 = _pad_idx_spread(edge_index_i2u[1], per_sc, N, ACC_N)
    het_src = jnp.stack([src0, src1]).reshape(NC, NCHUNK, CH)
    het_dst = jnp.stack([dst0, dst1]).reshape(NC, NCHUNK, CH)
    hom_src = _pad_idx(edge_index_hom[0], NC * per_sc, 0
                       ).reshape(NC, NCHUNK, CH)
    hom_dst = _pad_idx_spread(edge_index_hom[1], NC * per_sc, NHOM, ACC_H
                              ).reshape(NC, NCHUNK, CH)

    z_h = jnp.zeros((RT_H + PAD_R, PE_W), jnp.float32)
    z_n1 = jnp.zeros((RT_N + PAD_R + 2, C), jnp.float32)
    z_n = jnp.zeros((RT_N + PAD_R, C), jnp.float32)

    pe_pad = jnp.pad(PE, ((0, 0), (0, PE_W - PE_IN)))
    homp = _sc_hom(pe_pad, hom_src, hom_dst, z_h)
    homp_n = homp[:, :NHOM, :PE_IN]

    s1 = jnp.reshape(1.0 + l1['eps'], (1, 1))
    s2 = jnp.reshape(1.0 + l2['eps'], (1, 1))
    xall1, pe2 = pl.pallas_call(
        _tc_dense1,
        out_shape=(
            jax.ShapeDtypeStruct((NHOM, C), jnp.float32),
            jax.ShapeDtypeStruct((NHOM, PE_D), jnp.float32),
        ),
    )(PE, homp_n, x_user, x_item,
      s1, l1['gw1'], _row(l1['gb1']), l1['gw2'], _row(l1['gb2']),
      s2, l2['gw1'], _row(l2['gb1']), l2['gw2'], _row(l2['gb2']),
      l1['pew'][:C], l1['pew'][C:], _row(l1['peb']))

    sums1, cnt = _sc_mean1(xall1, het_src, het_dst, z_n1)
    sums1 = sums1[:, :N, :]
    cnt_n = cnt.reshape(NC, NS, 2 * C)[:, :, :RT_N].reshape(
        NC, ACC_N)[:, :N, None]

    y1 = pl.pallas_call(
        _tc_sage,
        out_shape=jax.ShapeDtypeStruct((NHOM, C), jnp.float32),
    )(sums1, cnt_n, xall1,
      l1['wl_u2i'], _row(l1['bl_u2i']), l1['wr_u2i'],
      l1['wl_i2u'], _row(l1['bl_i2u']), l1['wr_i2u'],
      _row(l1['lng_u']), _row(l1['lnb_u']),
      _row(l1['lng_i']), _row(l1['lnb_i']))

    xall2 = pl.pallas_call(
        _tc_pew,
        out_shape=jax.ShapeDtypeStruct((NHOM, C), jnp.float32),
    )(y1, pe2, l2['pew'][:C], l2['pew'][C:], _row(l2['peb']))

    sums2 = _sc_mean2(xall2, het_src, het_dst, z_n)[:, :N, :]

    out = pl.pallas_call(
        _tc_sage,
        out_shape=jax.ShapeDtypeStruct((NHOM, C), jnp.float32),
    )(sums2, cnt_n, xall2,
      l2['wl_u2i'], _row(l2['bl_u2i']), l2['wr_u2i'],
      l2['wl_i2u'], _row(l2['bl_i2u']), l2['wr_i2u'],
      _row(l2['lng_u']), _row(l2['lnb_u']),
      _row(l2['lng_i']), _row(l2['lnb_i']))
    return out
